# Initial kernel scaffold; baseline (speedup 1.0000x reference)
import jax
import jax.numpy as jnp
from jax.experimental import pallas as pl


def kernel(input_ids, table):
    raise NotImplementedError("write your pallas kernel here")


# speedup vs baseline: 1.3686x; 1.3686x over previous
import functools

import jax
import jax.numpy as jnp
from jax import lax
from jax.experimental import pallas as pl
from jax.experimental.pallas import tpu as pltpu
from jax.experimental.pallas import tpu_sc as plsc

EMBED_DIM = 512
NUM_TOKENS = 8 * 1024
NUM_WORKERS = 32
ROWS_PER_WORKER = NUM_TOKENS // NUM_WORKERS
CHUNK = 64
NUM_CHUNKS = ROWS_PER_WORKER // CHUNK

_mesh = plsc.VectorSubcoreMesh(core_axis_name="c", subcore_axis_name="s")


@functools.partial(
    pl.kernel,
    mesh=_mesh,
    out_type=jax.ShapeDtypeStruct((NUM_TOKENS, EMBED_DIM), jnp.float32),
    scratch_types=[
        pltpu.VMEM((NUM_CHUNKS, CHUNK), jnp.int32),
        pltpu.VMEM((CHUNK, EMBED_DIM), jnp.float32),
        pltpu.VMEM((CHUNK, EMBED_DIM), jnp.float32),
        pltpu.SemaphoreType.DMA,
        pltpu.SemaphoreType.DMA,
    ],
)
def _gather_kernel(idx_hbm, table_hbm, out_hbm, idx_v, buf0, buf1, sem0, sem1):
    wid = lax.axis_index("s") * 2 + lax.axis_index("c")
    base = wid * ROWS_PER_WORKER
    pltpu.sync_copy(idx_hbm.at[wid], idx_v)

    bufs = (buf0, buf1)
    sems = (sem0, sem1)
    copies = [pltpu.async_copy(table_hbm.at[idx_v.at[0]], bufs[0], sems[0])]
    for c in range(NUM_CHUNKS):
        if c + 1 < NUM_CHUNKS:
            copies.append(
                pltpu.async_copy(
                    table_hbm.at[idx_v.at[c + 1]],
                    bufs[(c + 1) % 2],
                    sems[(c + 1) % 2],
                )
            )
        copies[c].wait()
        pltpu.sync_copy(bufs[c % 2], out_hbm.at[pl.ds(base + c * CHUNK, CHUNK)])


def kernel(input_ids, table):
    ids = input_ids.reshape(-1).astype(jnp.int32)
    ids = ids.reshape(NUM_WORKERS, NUM_CHUNKS, CHUNK)
    flat = _gather_kernel(ids, table)
    embedding = flat.reshape(8, 1024, EMBED_DIM)
    ones = jnp.ones((8, 1024), dtype=jnp.float32)
    return (ones, embedding)


# speedup vs baseline: 1.3816x; 1.0095x over previous
import functools

import jax
import jax.numpy as jnp
from jax import lax
from jax.experimental import pallas as pl
from jax.experimental.pallas import tpu as pltpu
from jax.experimental.pallas import tpu_sc as plsc

EMBED_DIM = 512
NUM_TOKENS = 8 * 1024
NUM_WORKERS = 32
ROWS_PER_WORKER = NUM_TOKENS // NUM_WORKERS
CHUNK = 64
NUM_CHUNKS = ROWS_PER_WORKER // CHUNK
NUM_BUFS = 3

_mesh = plsc.VectorSubcoreMesh(core_axis_name="c", subcore_axis_name="s")


@functools.partial(
    pl.kernel,
    mesh=_mesh,
    out_type=jax.ShapeDtypeStruct((NUM_TOKENS, EMBED_DIM), jnp.float32),
    scratch_types=[
        pltpu.VMEM((NUM_CHUNKS, CHUNK), jnp.int32),
        *[pltpu.VMEM((CHUNK, EMBED_DIM), jnp.float32) for _ in range(NUM_BUFS)],
        *[pltpu.SemaphoreType.DMA for _ in range(2 * NUM_CHUNKS)],
    ],
)
def _gather_kernel(idx_hbm, table_hbm, out_hbm, idx_v, *rest):
    bufs = rest[:NUM_BUFS]
    gsems = rest[NUM_BUFS:NUM_BUFS + NUM_CHUNKS]
    wsems = rest[NUM_BUFS + NUM_CHUNKS:]

    wid = lax.axis_index("s") * 2 + lax.axis_index("c")
    base = wid * ROWS_PER_WORKER
    pltpu.sync_copy(idx_hbm.at[wid], idx_v)

    def gather(c):
        return pltpu.async_copy(
            table_hbm.at[idx_v.at[c]], bufs[c % NUM_BUFS], gsems[c]
        )

    def write(c):
        return pltpu.async_copy(
            bufs[c % NUM_BUFS], out_hbm.at[pl.ds(base + c * CHUNK, CHUNK)], wsems[c]
        )

    gc = [None] * NUM_CHUNKS
    wc = [None] * NUM_CHUNKS
    w_done = [False] * NUM_CHUNKS
    gc[0] = gather(0)
    gc[1] = gather(1)
    for c in range(NUM_CHUNKS):
        gc[c].wait()
        wc[c] = write(c)
        nxt = c + 2
        if nxt < NUM_CHUNKS:
            prev = nxt - NUM_BUFS
            if prev >= 0 and not w_done[prev]:
                wc[prev].wait()
                w_done[prev] = True
            gc[nxt] = gather(nxt)
    for c in range(NUM_CHUNKS):
        if not w_done[c]:
            wc[c].wait()


def kernel(input_ids, table):
    ids = input_ids.reshape(-1).astype(jnp.int32)
    ids = ids.reshape(NUM_WORKERS, NUM_CHUNKS, CHUNK)
    flat = _gather_kernel(ids, table)
    embedding = flat.reshape(8, 1024, EMBED_DIM)
    ones = jnp.ones((8, 1024), dtype=jnp.float32)
    return (ones, embedding)


# speedup vs baseline: 1.4051x; 1.0170x over previous
import functools

import jax
import jax.numpy as jnp
from jax import lax
from jax.experimental import pallas as pl
from jax.experimental.pallas import tpu as pltpu
from jax.experimental.pallas import tpu_sc as plsc

EMBED_DIM = 512
IDS_ROWS = 8
IDS_COLS = 1024
NUM_TOKENS = IDS_ROWS * IDS_COLS
NUM_WORKERS = 32
ROWS_PER_WORKER = NUM_TOKENS // NUM_WORKERS
WORKERS_PER_ROW = IDS_COLS // ROWS_PER_WORKER
CHUNK = 64
NUM_CHUNKS = ROWS_PER_WORKER // CHUNK
NUM_BUFS = 3

_mesh = plsc.VectorSubcoreMesh(core_axis_name="c", subcore_axis_name="s")


@functools.partial(
    pl.kernel,
    mesh=_mesh,
    out_type=(
        jax.ShapeDtypeStruct((NUM_TOKENS, EMBED_DIM), jnp.float32),
        jax.ShapeDtypeStruct((IDS_ROWS, IDS_COLS), jnp.float32),
    ),
    scratch_types=[
        pltpu.VMEM((ROWS_PER_WORKER,), jnp.int32),
        pltpu.VMEM((ROWS_PER_WORKER,), jnp.float32),
        *[pltpu.VMEM((CHUNK, EMBED_DIM), jnp.float32) for _ in range(NUM_BUFS)],
        *[pltpu.SemaphoreType.DMA for _ in range(2 * NUM_CHUNKS + 1)],
    ],
)
def _gather_kernel(idx_hbm, table_hbm, out_hbm, ones_hbm, idx_v, ones_v, *rest):
    bufs = rest[:NUM_BUFS]
    gsems = rest[NUM_BUFS:NUM_BUFS + NUM_CHUNKS]
    wsems = rest[NUM_BUFS + NUM_CHUNKS:NUM_BUFS + 2 * NUM_CHUNKS]
    osem = rest[NUM_BUFS + 2 * NUM_CHUNKS]

    wid = lax.axis_index("s") * 2 + lax.axis_index("c")
    base = wid * ROWS_PER_WORKER
    row = wid // WORKERS_PER_ROW
    col = (wid % WORKERS_PER_ROW) * ROWS_PER_WORKER
    pltpu.sync_copy(idx_hbm.at[row, pl.ds(col, ROWS_PER_WORKER)], idx_v)

    def gather(c):
        return pltpu.async_copy(
            table_hbm.at[idx_v.at[pl.ds(c * CHUNK, CHUNK)]],
            bufs[c % NUM_BUFS],
            gsems[c],
        )

    def write(c):
        return pltpu.async_copy(
            bufs[c % NUM_BUFS], out_hbm.at[pl.ds(base + c * CHUNK, CHUNK)], wsems[c]
        )

    gc = [None] * NUM_CHUNKS
    wc = [None] * NUM_CHUNKS
    w_done = [False] * NUM_CHUNKS
    gc[0] = gather(0)
    gc[1] = gather(1)

    one16 = jnp.full((16,), 1.0, dtype=jnp.float32)
    for i in range(ROWS_PER_WORKER // 16):
        ones_v[pl.ds(i * 16, 16)] = one16
    ones_cp = pltpu.async_copy(
        ones_v, ones_hbm.at[row, pl.ds(col, ROWS_PER_WORKER)], osem
    )

    for c in range(NUM_CHUNKS):
        gc[c].wait()
        wc[c] = write(c)
        nxt = c + 2
        if nxt < NUM_CHUNKS:
            prev = nxt - NUM_BUFS
            if prev >= 0 and not w_done[prev]:
                wc[prev].wait()
                w_done[prev] = True
            gc[nxt] = gather(nxt)
    for c in range(NUM_CHUNKS):
        if not w_done[c]:
            wc[c].wait()
    ones_cp.wait()


def kernel(input_ids, table):
    ids = input_ids.astype(jnp.int32)
    flat, ones = _gather_kernel(ids, table)
    embedding = flat.reshape(IDS_ROWS, IDS_COLS, EMBED_DIM)
    return (ones, embedding)


# speedup vs baseline: 1.4155x; 1.0074x over previous
import functools

import jax
import jax.numpy as jnp
from jax import lax
from jax.experimental import pallas as pl
from jax.experimental.pallas import tpu as pltpu
from jax.experimental.pallas import tpu_sc as plsc

EMBED_DIM = 512
IDS_ROWS = 8
IDS_COLS = 1024
NUM_TOKENS = IDS_ROWS * IDS_COLS
NUM_WORKERS = 32
ROWS_PER_WORKER = NUM_TOKENS // NUM_WORKERS
WORKERS_PER_ROW = IDS_COLS // ROWS_PER_WORKER
CHUNK = 32
NUM_CHUNKS = ROWS_PER_WORKER // CHUNK
NUM_BUFS = 6
GDEPTH = 3

_mesh = plsc.VectorSubcoreMesh(core_axis_name="c", subcore_axis_name="s")


@functools.partial(
    pl.kernel,
    mesh=_mesh,
    out_type=(
        jax.ShapeDtypeStruct((NUM_TOKENS, EMBED_DIM), jnp.float32),
        jax.ShapeDtypeStruct((IDS_ROWS, IDS_COLS), jnp.float32),
    ),
    scratch_types=[
        pltpu.VMEM((ROWS_PER_WORKER,), jnp.int32),
        pltpu.VMEM((ROWS_PER_WORKER,), jnp.float32),
        *[pltpu.VMEM((CHUNK, EMBED_DIM), jnp.float32) for _ in range(NUM_BUFS)],
        *[pltpu.SemaphoreType.DMA for _ in range(2 * NUM_CHUNKS + 1)],
    ],
)
def _gather_kernel(idx_hbm, table_hbm, out_hbm, ones_hbm, idx_v, ones_v, *rest):
    bufs = rest[:NUM_BUFS]
    gsems = rest[NUM_BUFS:NUM_BUFS + NUM_CHUNKS]
    wsems = rest[NUM_BUFS + NUM_CHUNKS:NUM_BUFS + 2 * NUM_CHUNKS]
    osem = rest[NUM_BUFS + 2 * NUM_CHUNKS]

    wid = lax.axis_index("s") * 2 + lax.axis_index("c")
    base = wid * ROWS_PER_WORKER
    row = wid // WORKERS_PER_ROW
    col = (wid % WORKERS_PER_ROW) * ROWS_PER_WORKER
    pltpu.sync_copy(idx_hbm.at[row, pl.ds(col, ROWS_PER_WORKER)], idx_v)

    def gather(c):
        return pltpu.async_copy(
            table_hbm.at[idx_v.at[pl.ds(c * CHUNK, CHUNK)]],
            bufs[c % NUM_BUFS],
            gsems[c],
        )

    def write(c):
        return pltpu.async_copy(
            bufs[c % NUM_BUFS], out_hbm.at[pl.ds(base + c * CHUNK, CHUNK)], wsems[c]
        )

    gc = [None] * NUM_CHUNKS
    wc = [None] * NUM_CHUNKS
    w_done = [False] * NUM_CHUNKS
    for c in range(GDEPTH):
        gc[c] = gather(c)

    one16 = jnp.full((16,), 1.0, dtype=jnp.float32)
    for i in range(ROWS_PER_WORKER // 16):
        ones_v[pl.ds(i * 16, 16)] = one16
    ones_cp = pltpu.async_copy(
        ones_v, ones_hbm.at[row, pl.ds(col, ROWS_PER_WORKER)], osem
    )

    for c in range(NUM_CHUNKS):
        gc[c].wait()
        wc[c] = write(c)
        nxt = c + GDEPTH
        if nxt < NUM_CHUNKS:
            prev = nxt - NUM_BUFS
            if prev >= 0 and not w_done[prev]:
                wc[prev].wait()
                w_done[prev] = True
            gc[nxt] = gather(nxt)
    for c in range(NUM_CHUNKS):
        if not w_done[c]:
            wc[c].wait()
    ones_cp.wait()


def kernel(input_ids, table):
    ids = input_ids.astype(jnp.int32)
    flat, ones = _gather_kernel(ids, table)
    embedding = flat.reshape(IDS_ROWS, IDS_COLS, EMBED_DIM)
    return (ones, embedding)


# speedup vs baseline: 1.4390x; 1.0166x over previous
import functools

import jax
import jax.numpy as jnp
from jax import lax
from jax.experimental import pallas as pl
from jax.experimental.pallas import tpu as pltpu
from jax.experimental.pallas import tpu_sc as plsc

EMBED_DIM = 512
IDS_ROWS = 8
IDS_COLS = 1024
NUM_TOKENS = IDS_ROWS * IDS_COLS
NUM_WORKERS = 32
ROWS_PER_WORKER = NUM_TOKENS // NUM_WORKERS
WORKERS_PER_ROW = IDS_COLS // ROWS_PER_WORKER
CHUNK = 32
NUM_CHUNKS = ROWS_PER_WORKER // CHUNK
NUM_BUFS = 7
GDEPTH = 4

_mesh = plsc.VectorSubcoreMesh(core_axis_name="c", subcore_axis_name="s")


@functools.partial(
    pl.kernel,
    mesh=_mesh,
    out_type=(
        jax.ShapeDtypeStruct((NUM_TOKENS, EMBED_DIM), jnp.float32),
        jax.ShapeDtypeStruct((IDS_ROWS, IDS_COLS), jnp.float32),
    ),
    scratch_types=[
        pltpu.VMEM((ROWS_PER_WORKER,), jnp.int32),
        pltpu.VMEM((ROWS_PER_WORKER,), jnp.float32),
        *[pltpu.VMEM((CHUNK, EMBED_DIM), jnp.float32) for _ in range(NUM_BUFS)],
        *[pltpu.SemaphoreType.DMA for _ in range(2 * NUM_CHUNKS + 1)],
    ],
)
def _gather_kernel(idx_hbm, table_hbm, out_hbm, ones_hbm, idx_v, ones_v, *rest):
    bufs = rest[:NUM_BUFS]
    gsems = rest[NUM_BUFS:NUM_BUFS + NUM_CHUNKS]
    wsems = rest[NUM_BUFS + NUM_CHUNKS:NUM_BUFS + 2 * NUM_CHUNKS]
    osem = rest[NUM_BUFS + 2 * NUM_CHUNKS]

    wid = lax.axis_index("s") * 2 + lax.axis_index("c")
    base = wid * ROWS_PER_WORKER
    row = wid // WORKERS_PER_ROW
    col = (wid % WORKERS_PER_ROW) * ROWS_PER_WORKER
    pltpu.sync_copy(idx_hbm.at[row, pl.ds(col, ROWS_PER_WORKER)], idx_v)

    def gather(c):
        return pltpu.async_copy(
            table_hbm.at[idx_v.at[pl.ds(c * CHUNK, CHUNK)]],
            bufs[c % NUM_BUFS],
            gsems[c],
        )

    def write(c):
        return pltpu.async_copy(
            bufs[c % NUM_BUFS], out_hbm.at[pl.ds(base + c * CHUNK, CHUNK)], wsems[c]
        )

    gc = [None] * NUM_CHUNKS
    wc = [None] * NUM_CHUNKS
    w_done = [False] * NUM_CHUNKS
    for c in range(GDEPTH):
        gc[c] = gather(c)

    one16 = jnp.full((16,), 1.0, dtype=jnp.float32)
    for i in range(ROWS_PER_WORKER // 16):
        ones_v[pl.ds(i * 16, 16)] = one16
    ones_cp = pltpu.async_copy(
        ones_v, ones_hbm.at[row, pl.ds(col, ROWS_PER_WORKER)], osem
    )

    for c in range(NUM_CHUNKS):
        gc[c].wait()
        wc[c] = write(c)
        nxt = c + GDEPTH
        if nxt < NUM_CHUNKS:
            prev = nxt - NUM_BUFS
            if prev >= 0 and not w_done[prev]:
                wc[prev].wait()
                w_done[prev] = True
            gc[nxt] = gather(nxt)
    for c in range(NUM_CHUNKS):
        if not w_done[c]:
            wc[c].wait()
    ones_cp.wait()


def kernel(input_ids, table):
    ids = input_ids.astype(jnp.int32)
    flat, ones = _gather_kernel(ids, table)
    embedding = flat.reshape(IDS_ROWS, IDS_COLS, EMBED_DIM)
    return (ones, embedding)
